# Initial kernel scaffold; baseline (speedup 1.0000x reference)
import jax
import jax.numpy as jnp
from jax.experimental import pallas as pl


def kernel(tensor):
    raise NotImplementedError("write your pallas kernel here")


# speedup vs baseline: 1.8587x; 1.8587x over previous
import functools

import jax
import jax.numpy as jnp
from jax import lax
from jax.experimental import pallas as pl
from jax.experimental.pallas import tpu as pltpu
from jax.experimental.pallas import tpu_sc as plsc

_ROWS, _COLS = 8192, 4096
_NC, _NS, _L = 2, 16, 16
_NW = _NC * _NS
_CPW = _COLS // _NW
_G = _CPW // _L
_CHUNK = 256
_NCHUNK = _ROWS // _CHUNK

_mesh = plsc.VectorSubcoreMesh(core_axis_name="c", subcore_axis_name="s")


@functools.partial(
    pl.kernel,
    out_type=jax.ShapeDtypeStruct((_ROWS, _COLS), jnp.float32),
    mesh=_mesh,
    scratch_types=[pltpu.VMEM((_CHUNK, _CPW), jnp.float32)],
)
def _sc_cumsum(in_hbm, out_hbm, buf):
    wid = lax.axis_index("s") * _NC + lax.axis_index("c")
    c0 = wid * _CPW

    def chunk_body(i, carry):
        r0 = i * _CHUNK
        pltpu.sync_copy(in_hbm.at[pl.ds(r0, _CHUNK), pl.ds(c0, _CPW)], buf)

        def row_body(r, c):
            new = []
            for g in range(_G):
                v = buf[r, pl.ds(g * _L, _L)]
                cg = c[g] + v
                buf[r, pl.ds(g * _L, _L)] = cg
                new.append(cg)
            return tuple(new)

        carry = lax.fori_loop(0, _CHUNK, row_body, carry)
        pltpu.sync_copy(buf, out_hbm.at[pl.ds(r0, _CHUNK), pl.ds(c0, _CPW)])
        return carry

    zero = jnp.zeros((_L,), jnp.float32)
    lax.fori_loop(0, _NCHUNK, chunk_body, tuple(zero for _ in range(_G)))


def kernel(tensor):
    return _sc_cumsum(tensor)
